# baseline (speedup 1.0000x reference)
import functools

import jax
import jax.numpy as jnp
import numpy as np
from jax import lax
from jax.experimental import pallas as pl
from jax.experimental.pallas import tpu as pltpu
from jax.experimental.pallas import tpu_sc as plsc

N_NODES = 10000
N_EDGES = 160000
HIDDEN = 128
BINS = 128
NUM_GRAPHS = 128
VMIN, VMAX = 0.0, 8.0

E_HALF = N_EDGES // 2
BE = 640
BN = 400
GEH = E_HALF // BE
GN = N_NODES // BN

SC_CORES = 2
SC_TILES = 16
SC_WORKERS = SC_CORES * SC_TILES
CHUNK = 40

_mesh = functools.partial(
    plsc.VectorSubcoreMesh, core_axis_name="c", subcore_axis_name="s")


def _silu(x):
    return x * jax.nn.sigmoid(x)


def _sc_gather(xcat, gidx3):
    n_chunks = gidx3.shape[0]
    per_w = n_chunks // SC_WORKERS
    D = xcat.shape[1]
    dt = xcat.dtype

    @functools.partial(
        pl.kernel,
        out_type=jax.ShapeDtypeStruct((n_chunks * CHUNK, D), dt),
        mesh=_mesh(),
        scratch_types=[
            pltpu.VMEM((per_w, 1, CHUNK), jnp.int32),
            pltpu.VMEM((CHUNK, D), dt),
            pltpu.VMEM((CHUNK, D), dt),
            pltpu.SemaphoreType.DMA,
            pltpu.SemaphoreType.DMA,
        ],
    )
    def k(x_hbm, idx_hbm, out_hbm, idx_v3, buf0, buf1, sem0, sem1):
        wid = lax.axis_index("s") * SC_CORES + lax.axis_index("c")
        chunk0 = wid * per_w
        pltpu.sync_copy(idx_hbm.at[pl.ds(chunk0, per_w)], idx_v3)

        def fire(i, buf, sem):
            pltpu.async_copy(x_hbm.at[idx_v3.at[i, 0]], buf, sem)

        def drain(i, buf, sem):
            pltpu.make_async_copy(x_hbm.at[idx_v3.at[i, 0]], buf, sem).wait()

        def wb(i, buf):
            pltpu.sync_copy(buf,
                            out_hbm.at[pl.ds((chunk0 + i) * CHUNK, CHUNK)])

        fire(0, buf0, sem0)
        fire(1, buf1, sem1)

        def body(r, carry):
            i0 = 2 * r
            drain(i0, buf0, sem0)
            wb(i0, buf0)

            @pl.when(i0 + 2 < per_w)
            def _():
                fire(i0 + 2, buf0, sem0)

            drain(i0 + 1, buf1, sem1)
            wb(i0 + 1, buf1)

            @pl.when(i0 + 3 < per_w)
            def _():
                fire(i0 + 3, buf1, sem1)

            return carry

        lax.fori_loop(0, per_w // 2, body, 0)
        if per_w % 2 == 1:
            drain(per_w - 1, buf0, sem0)
            wb(per_w - 1, buf0)

    return k(xcat, gidx3)


ROWS_T = 624
ROWS_TAIL = N_NODES - ROWS_T * SC_TILES


def _sc_scatter(cart_msg, mat_msg, sidx3, zeros_tile):
    n_chunks = sidx3.shape[0]
    per_t = n_chunks // SC_TILES

    @functools.partial(
        pl.kernel,
        out_type=jax.ShapeDtypeStruct((2, N_NODES, HIDDEN), jnp.float32),
        mesh=_mesh(),
        scratch_types=[
            pltpu.VMEM((per_t, 1, CHUNK), jnp.int32),
            pltpu.VMEM((CHUNK, HIDDEN), jnp.float32),
            pltpu.VMEM((CHUNK, HIDDEN), jnp.float32),
            pltpu.VMEM_SHARED((N_NODES, HIDDEN), jnp.float32),
            pltpu.SemaphoreType.DMA,
            pltpu.SemaphoreType.DMA,
        ],
    )
    def k(cm_hbm, mm_hbm, idx_hbm, z_hbm, out_hbm,
          idx_v3, buf0, buf1, acc_sh, sem0, sem1):
        c = lax.axis_index("c")
        s = lax.axis_index("s")
        chunk0 = s * per_t
        pltpu.sync_copy(z_hbm.at[pl.ds(0, ROWS_T)],
                        acc_sh.at[pl.ds(s * ROWS_T, ROWS_T)])

        @pl.when(s == SC_TILES - 1)
        def _():
            pltpu.sync_copy(z_hbm.at[pl.ds(0, ROWS_TAIL)],
                            acc_sh.at[pl.ds(SC_TILES * ROWS_T, ROWS_TAIL)])

        pltpu.sync_copy(idx_hbm.at[pl.ds(chunk0, per_t)], idx_v3)
        plsc.subcore_barrier()

        def fire(i, buf, sem):
            row0 = (chunk0 + i) * CHUNK

            @pl.when(c == 0)
            def _():
                pltpu.async_copy(cm_hbm.at[pl.ds(row0, CHUNK)], buf, sem)

            @pl.when(c == 1)
            def _():
                pltpu.async_copy(mm_hbm.at[pl.ds(row0, CHUNK)], buf, sem)

        def drain(buf, sem):
            pltpu.make_async_copy(cm_hbm.at[pl.ds(0, CHUNK)], buf, sem).wait()

        def add(i, buf):
            pltpu.sync_copy(buf, acc_sh.at[idx_v3.at[i, 0]], add=True)

        fire(0, buf0, sem0)
        fire(1, buf1, sem1)

        def body(r, carry):
            i0 = 2 * r
            drain(buf0, sem0)
            add(i0, buf0)

            @pl.when(i0 + 2 < per_t)
            def _():
                fire(i0 + 2, buf0, sem0)

            drain(buf1, sem1)
            add(i0 + 1, buf1)

            @pl.when(i0 + 3 < per_t)
            def _():
                fire(i0 + 3, buf1, sem1)

            return carry

        lax.fori_loop(0, per_t // 2, body, 0)
        if per_t % 2 == 1:
            drain(buf0, sem0)
            add(per_t - 1, buf0)
        plsc.subcore_barrier()

        def wb(o):
            pltpu.sync_copy(acc_sh.at[pl.ds(s * ROWS_T, ROWS_T)],
                            out_hbm.at[o, pl.ds(s * ROWS_T, ROWS_T)])

            @pl.when(s == SC_TILES - 1)
            def _():
                pltpu.sync_copy(
                    acc_sh.at[pl.ds(SC_TILES * ROWS_T, ROWS_TAIL)],
                    out_hbm.at[o, pl.ds(SC_TILES * ROWS_T, ROWS_TAIL)])

        @pl.when(c == 0)
        def _():
            wb(0)

        @pl.when(c == 1)
        def _():
            wb(1)

    return k(cart_msg, mat_msg, sidx3, zeros_tile)


def _embed_kernel(t_ref, emb_ref, h_ref):
    t = t_ref[0, 0, :]
    oh = (t[:, None] == lax.broadcasted_iota(jnp.int32, (BN, 128), 1))
    h_ref[...] = jnp.dot(oh.astype(jnp.float32), emb_ref[...],
                         preferred_element_type=jnp.float32)


def _embed(x_types, emb_pad):
    t3 = x_types.astype(jnp.int32).reshape(GN, 1, BN)
    return pl.pallas_call(
        _embed_kernel,
        grid=(GN,),
        in_specs=[
            pl.BlockSpec((1, 1, BN), lambda i: (i, 0, 0)),
            pl.BlockSpec((128, 128), lambda i: (0, 0)),
        ],
        out_specs=pl.BlockSpec((BN, HIDDEN), lambda i: (i, 0)),
        out_shape=jax.ShapeDtypeStruct((N_NODES, HIDDEN), jnp.float32),
    )(t3, emb_pad)


def _rbf_kernel(a_ref, w1_ref, b1_ref, w2_ref, b2_ref, e_ref):
    gamma = (BINS - 1) / (VMAX - VMIN)
    a = a_ref[...]
    d = jnp.sqrt(jnp.sum(a * a, axis=1, keepdims=True))
    centers = (lax.broadcasted_iota(jnp.int32, (1, BINS), 1).astype(jnp.float32)
               * ((VMAX - VMIN) / (BINS - 1)) + VMIN)
    rbf = jnp.exp(-gamma * (d - centers) ** 2)
    h = jnp.dot(rbf, w1_ref[...], preferred_element_type=jnp.float32) + b1_ref[...]
    sp = jnp.where(h > 20.0, h, jnp.log(1.0 + jnp.exp(jnp.minimum(h, 20.0))))
    e_ref[...] = (jnp.dot(sp, w2_ref[...], preferred_element_type=jnp.float32)
                  + b2_ref[...])


def _edge_feats(ea_pad, w1, b1, w2, b2):
    return pl.pallas_call(
        _rbf_kernel,
        grid=(GEH,),
        in_specs=[
            pl.BlockSpec((BE, 8), lambda i: (i, 0)),
            pl.BlockSpec((BINS, HIDDEN), lambda i: (0, 0)),
            pl.BlockSpec((1, HIDDEN), lambda i: (0, 0)),
            pl.BlockSpec((HIDDEN, HIDDEN), lambda i: (0, 0)),
            pl.BlockSpec((1, HIDDEN), lambda i: (0, 0)),
        ],
        out_specs=pl.BlockSpec((BE, HIDDEN), lambda i: (i, 0)),
        out_shape=jax.ShapeDtypeStruct((E_HALF, HIDDEN), jnp.float32),
    )(ea_pad, w1, b1, w2, b2)


def _edge_kernel(xgd_ref, xgs_ref, ec_ref, e0_ref,
                 wcart_ref, bcart_ref, g2_ref, a2_ref, b2c_ref, bne_ref,
                 wqk_ref, bqk_ref, wkk_ref, bk_ref,
                 we_ref, bea_ref, wv_ref, bv_ref, m3_ref,
                 wml_ref, bml_ref, aln_ref, mln_ref,
                 enew_ref, cmsg_ref, mmsg_ref):
    f32 = jnp.float32
    xcd = xgd_ref[:, 0:HIDDEN]
    xmd = xgd_ref[:, HIDDEN:2 * HIDDEN]
    xcs = xgs_ref[:, 0:HIDDEN]
    xms = xgs_ref[:, HIDDEN:2 * HIDDEN]
    ec = ec_ref[...]
    e0 = e0_ref[...]

    z = jnp.concatenate([xcd, xcs, ec], axis=1)
    pre = jnp.dot(z, wcart_ref[...], preferred_element_type=f32) + bcart_ref[...]
    sg = _silu(pre[:, 0:HIDDEN])
    sm = _silu(pre[:, HIDDEN:2 * HIDDEN])
    e_ij = (jnp.dot(sg, g2_ref[...], preferred_element_type=f32)
            + b2c_ref[:, 0:HIDDEN])
    m = (jnp.dot(sm, a2_ref[...], preferred_element_type=f32)
         + b2c_ref[:, HIDDEN:2 * HIDDEN])
    cmsg_ref[...] = jax.nn.sigmoid(e_ij) * m
    enew_ref[...] = _silu(e_ij * bne_ref[0:1, :] + bne_ref[1:2, :]) + ec

    qk = jnp.dot(xmd, wqk_ref[...], preferred_element_type=f32) + bqk_ref[...]
    q_d = qk[:, 0:HIDDEN]
    k_d = qk[:, HIDDEN:2 * HIDDEN]
    k_s = jnp.dot(xms, wkk_ref[...], preferred_element_type=f32) + bk_ref[...]
    ea = jnp.dot(e0, we_ref[...], preferred_element_type=f32) + bea_ref[...]
    scale = 1.0 / np.sqrt(3.0 * HIDDEN).astype(np.float32)
    alpha = jnp.concatenate([q_d * k_d, q_d * k_s, q_d * ea], axis=1) * scale
    mu = jnp.mean(alpha, axis=1, keepdims=True)
    var = jnp.mean((alpha - mu) ** 2, axis=1, keepdims=True)
    aln = ((alpha - mu) / jnp.sqrt(var + 1e-5) * aln_ref[0:1, :]
           + aln_ref[1:2, :])
    sig = jax.nn.sigmoid(aln)
    xm2 = jnp.concatenate([xmd, xms], axis=1)
    vpart = jnp.dot(xm2, wv_ref[...], preferred_element_type=f32) + bv_ref[...]
    msg = (vpart + jnp.dot(ea, m3_ref[...], preferred_element_type=f32)) * sig
    h2 = jnp.dot(msg, wml_ref[...], preferred_element_type=f32) + bml_ref[...]
    mu2 = jnp.mean(h2, axis=1, keepdims=True)
    var2 = jnp.mean((h2 - mu2) ** 2, axis=1, keepdims=True)
    mmsg_ref[...] = ((h2 - mu2) / jnp.sqrt(var2 + 1e-5) * mln_ref[0:1, :]
                     + mln_ref[1:2, :])


def _edge_layer(xg, ec, e0, w):
    full = lambda shape: pl.BlockSpec(shape, lambda i: (0, 0))
    return pl.pallas_call(
        _edge_kernel,
        grid=(GEH,),
        in_specs=[
            pl.BlockSpec((BE, 2 * HIDDEN), lambda i: (i, 0)),
            pl.BlockSpec((BE, 2 * HIDDEN), lambda i: (i + GEH, 0)),
            pl.BlockSpec((BE, HIDDEN), lambda i: (i, 0)),
            pl.BlockSpec((BE, HIDDEN), lambda i: (i, 0)),
            full((3 * HIDDEN, 2 * HIDDEN)), full((1, 2 * HIDDEN)),
            full((HIDDEN, HIDDEN)), full((HIDDEN, HIDDEN)),
            full((1, 2 * HIDDEN)), full((2, HIDDEN)),
            full((HIDDEN, 2 * HIDDEN)), full((1, 2 * HIDDEN)),
            full((HIDDEN, HIDDEN)), full((1, HIDDEN)),
            full((HIDDEN, HIDDEN)), full((1, HIDDEN)),
            full((2 * HIDDEN, 3 * HIDDEN)), full((1, 3 * HIDDEN)),
            full((HIDDEN, 3 * HIDDEN)),
            full((3 * HIDDEN, HIDDEN)), full((1, HIDDEN)),
            full((2, 3 * HIDDEN)), full((2, HIDDEN)),
        ],
        out_specs=[
            pl.BlockSpec((BE, HIDDEN), lambda i: (i, 0)),
            pl.BlockSpec((BE, HIDDEN), lambda i: (i, 0)),
            pl.BlockSpec((BE, HIDDEN), lambda i: (i, 0)),
        ],
        out_shape=[
            jax.ShapeDtypeStruct((E_HALF, HIDDEN), jnp.float32),
            jax.ShapeDtypeStruct((E_HALF, HIDDEN), jnp.float32),
            jax.ShapeDtypeStruct((E_HALF, HIDDEN), jnp.float32),
        ],
    )(xg, xg, ec, e0,
      w["Wcart"], w["bcart"], w["G2"], w["A2"], w["b2c"], w["bne"],
      w["Wqk"], w["bqk"], w["Wkk"], w["bk"],
      w["We"], w["bea"], w["WV"], w["bV"], w["M3"],
      w["Wml"], w["bml"], w["aln"], w["mln"])


def _node_kernel(xc_ref, xm_ref, aggc0_ref, aggm0_ref, aggc1_ref, aggm1_ref,
                 bnx_ref, ws_ref, bs_ref, wb_ref, wmix_ref, bmix_ref,
                 xcn_ref, xmn_ref):
    f32 = jnp.float32
    xc_old = xc_ref[...]
    xm_old = xm_ref[...]
    agg = aggc0_ref[0] + aggc1_ref[0]
    out = aggm0_ref[0] + aggm1_ref[0]
    x_c = _silu(agg * bnx_ref[0:1, :] + bnx_ref[1:2, :]) + xc_old
    x_r = jnp.dot(xm_old, ws_ref[...], preferred_element_type=f32) + bs_ref[0:1, :]
    bpre = (jnp.sum(out * wb_ref[0:1, :], axis=1, keepdims=True)
            + jnp.sum(x_r * wb_ref[1:2, :], axis=1, keepdims=True)
            + wb_ref[2, 0])
    beta = jax.nn.sigmoid(bpre)
    x_m = beta * x_r + (1.0 - beta) * out
    z = jnp.concatenate([x_c, x_m], axis=1)
    gate = jax.nn.sigmoid(
        jnp.dot(z, wmix_ref[...], preferred_element_type=f32) + bmix_ref[...])
    x_out = gate * x_c + (1.0 - gate) * x_m
    xcn_ref[...] = x_out + x_c
    xmn_ref[...] = x_out + x_m


def _node_layer(x_cart, x_mat, agg2a, agg2b, w):
    full = lambda shape: pl.BlockSpec(shape, lambda i: tuple(0 for _ in shape))
    return pl.pallas_call(
        _node_kernel,
        grid=(GN,),
        in_specs=[
            pl.BlockSpec((BN, HIDDEN), lambda i: (i, 0)),
            pl.BlockSpec((BN, HIDDEN), lambda i: (i, 0)),
            pl.BlockSpec((1, BN, HIDDEN), lambda i: (0, i, 0)),
            pl.BlockSpec((1, BN, HIDDEN), lambda i: (1, i, 0)),
            pl.BlockSpec((1, BN, HIDDEN), lambda i: (0, i, 0)),
            pl.BlockSpec((1, BN, HIDDEN), lambda i: (1, i, 0)),
            full((2, HIDDEN)),
            full((HIDDEN, HIDDEN)), full((1, HIDDEN)),
            full((3, HIDDEN)),
            full((2 * HIDDEN, HIDDEN)), full((1, HIDDEN)),
        ],
        out_specs=[
            pl.BlockSpec((BN, HIDDEN), lambda i: (i, 0)),
            pl.BlockSpec((BN, HIDDEN), lambda i: (i, 0)),
        ],
        out_shape=[
            jax.ShapeDtypeStruct((N_NODES, HIDDEN), jnp.float32),
            jax.ShapeDtypeStruct((N_NODES, HIDDEN), jnp.float32),
        ],
    )(x_cart, x_mat, agg2a, agg2a, agg2b, agg2b,
      w["bnx"], w["Ws"], w["bs"], w["wb"], w["Wmix"], w["bmix"])


def _readout_kernel(xc_ref, xm_ref, b_ref,
                    w1_ref, b1_ref, w2r_ref, misc_ref,
                    out_ref, sums_ref):
    i = pl.program_id(0)

    @pl.when(i == 0)
    def _():
        sums_ref[...] = jnp.zeros_like(sums_ref)

    xf = (xc_ref[...] + xm_ref[...]) * 0.5
    bids = b_ref[0, 0, :]
    oh = (lax.broadcasted_iota(jnp.int32, (NUM_GRAPHS, BN), 0)
          == bids[None, :]).astype(jnp.float32)
    ones = jnp.ones((BN, HIDDEN), jnp.float32)
    xa = jnp.concatenate([xf, ones], axis=1)
    sums_ref[...] += jnp.dot(oh, xa, preferred_element_type=jnp.float32)

    @pl.when(i == GN - 1)
    def _():
        sums = sums_ref[:, 0:HIDDEN]
        counts = sums_ref[:, HIDDEN:HIDDEN + 1]
        feats = sums / jnp.maximum(counts, 1.0)
        h = _silu(jnp.dot(feats, w1_ref[...], preferred_element_type=jnp.float32)
                  + b1_ref[...])
        o = lax.dot_general(w2r_ref[...], h, (((1,), (1,)), ((), ())))
        out_ref[...] = o + misc_ref[0, 0]


def _readout(x_cart, x_mat, batch_ids, w1, b1, w2r, misc):
    b3 = batch_ids.astype(jnp.int32).reshape(GN, 1, BN)
    full = lambda shape: pl.BlockSpec(shape, lambda i: tuple(0 for _ in shape))
    return pl.pallas_call(
        _readout_kernel,
        grid=(GN,),
        in_specs=[
            pl.BlockSpec((BN, HIDDEN), lambda i: (i, 0)),
            pl.BlockSpec((BN, HIDDEN), lambda i: (i, 0)),
            pl.BlockSpec((1, 1, BN), lambda i: (i, 0, 0)),
            full((HIDDEN, HIDDEN)), full((1, HIDDEN)),
            full((1, HIDDEN)), full((1, HIDDEN)),
        ],
        out_specs=pl.BlockSpec((1, NUM_GRAPHS), lambda i: (0, 0)),
        out_shape=jax.ShapeDtypeStruct((1, NUM_GRAPHS), jnp.float32),
        scratch_shapes=[pltpu.VMEM((NUM_GRAPHS, 2 * HIDDEN), jnp.float32)],
    )(x_cart, x_mat, b3, w1, b1, w2r, misc)


def _fold_layer(lp):
    r1 = lambda b: b.reshape(1, -1)
    Wmu = lp["mat_msg_update"]["W"]
    M1, M2, M3 = Wmu[0:128], Wmu[128:256], Wmu[256:384]
    Wv, bv = lp["mat_v"]["W"], lp["mat_v"]["b"]
    Wbeta = lp["mat_beta"]["W"][:, 0]
    wb = jnp.stack([
        Wbeta[0:128] + Wbeta[256:384],
        Wbeta[128:256] - Wbeta[256:384],
        jnp.full((128,), lp["mat_beta"]["b"][0]),
    ])
    return {
        "Wcart": jnp.concatenate([lp["cart_gate1"]["W"],
                                  lp["cart_aggr1"]["W"]], axis=1),
        "bcart": jnp.concatenate([lp["cart_gate1"]["b"],
                                  lp["cart_aggr1"]["b"]]).reshape(1, -1),
        "G2": lp["cart_gate2"]["W"], "A2": lp["cart_aggr2"]["W"],
        "b2c": jnp.concatenate([lp["cart_gate2"]["b"],
                                lp["cart_aggr2"]["b"]]).reshape(1, -1),
        "bne": jnp.stack([lp["cart_bn_e"]["w"], lp["cart_bn_e"]["b"]]),
        "Wqk": jnp.concatenate([lp["mat_q"]["W"], lp["mat_k"]["W"]], axis=1),
        "bqk": jnp.concatenate([lp["mat_q"]["b"],
                                lp["mat_k"]["b"]]).reshape(1, -1),
        "Wkk": lp["mat_k"]["W"], "bk": r1(lp["mat_k"]["b"]),
        "We": lp["mat_edge"]["W"], "bea": r1(lp["mat_edge"]["b"]),
        "WV": jnp.concatenate([Wv @ M1, Wv @ M2], axis=0),
        "bV": (bv @ M1 + bv @ M2 + lp["mat_msg_update"]["b"]).reshape(1, -1),
        "M3": M3,
        "Wml": lp["mat_msg_lin"]["W"], "bml": r1(lp["mat_msg_lin"]["b"]),
        "aln": jnp.stack([lp["mat_alpha_ln"]["w"], lp["mat_alpha_ln"]["b"]]),
        "mln": jnp.stack([lp["mat_msg_ln"]["w"], lp["mat_msg_ln"]["b"]]),
        "bnx": jnp.stack([lp["cart_bn_x"]["w"], lp["cart_bn_x"]["b"]]),
        "Ws": lp["mat_skip"]["W"], "bs": r1(lp["mat_skip"]["b"]),
        "wb": wb,
        "Wmix": lp["mix_gate"]["W"], "bmix": r1(lp["mix_gate"]["b"]),
    }


def kernel(edge_attr, params, x_types, edge_index, batch_ids):
    src = edge_index[0].astype(jnp.int32)
    dst = edge_index[1].astype(jnp.int32)

    emb_pad = jnp.zeros((128, HIDDEN), jnp.float32).at[:119].set(
        params["atom_emb"])
    h = _embed(x_types, emb_pad)

    e0s, gidxs, sidxs = [], [], []
    for hh in range(2):
        sl = slice(hh * E_HALF, (hh + 1) * E_HALF)
        ea_pad = jnp.concatenate(
            [edge_attr[sl], jnp.zeros((E_HALF, 5), jnp.float32)], axis=1)
        e0s.append(_edge_feats(ea_pad, params["rbf_lin1"]["W"],
                               params["rbf_lin1"]["b"].reshape(1, -1),
                               params["rbf_lin2"]["W"],
                               params["rbf_lin2"]["b"].reshape(1, -1)))
        gidxs.append(jnp.concatenate([dst[sl], src[sl]]).reshape(-1, 1, CHUNK))
        sidxs.append(dst[sl].reshape(-1, 1, CHUNK))
    zeros_tile = jnp.zeros((ROWS_T, HIDDEN), jnp.float32)

    x_cart, x_mat = h, h
    ec = list(e0s)
    gidx_full = jnp.concatenate(
        [dst[:E_HALF], src[:E_HALF], dst[E_HALF:], src[E_HALF:]]
    ).reshape(-1, 1, CHUNK)
    sidx_full = dst.reshape(-1, 1, CHUNK)
    for lp in params["layers"]:
        w = _fold_layer(lp)
        xcat = jnp.concatenate([x_cart, x_mat], axis=1)
        xg_full = _sc_gather(xcat, gidx_full)
        aggs, ec_new = [], []
        for hh in range(2):
            xg = lax.slice_in_dim(xg_full, hh * 2 * E_HALF,
                                  (hh + 1) * 2 * E_HALF)
            ec_h, cmsg, mmsg = _edge_layer(xg, ec[hh], e0s[hh], w)
            aggs.append(_sc_scatter(cmsg, mmsg, sidxs[hh], zeros_tile))
            ec_new.append(ec_h)
        ec = ec_new
        x_cart, x_mat = _node_layer(x_cart, x_mat, aggs[0], aggs[1], w)

    out = _readout(x_cart, x_mat, batch_ids,
                   params["fc1"]["W"], params["fc1"]["b"].reshape(1, -1),
                   params["fc2"]["W"].T.reshape(1, -1),
                   jnp.full((1, HIDDEN), params["fc2"]["b"][0]))
    return out.reshape(NUM_GRAPHS)


# speedup vs baseline: 1.3076x; 1.3076x over previous
import functools

import jax
import jax.numpy as jnp
import numpy as np
from jax import lax
from jax.experimental import pallas as pl
from jax.experimental.pallas import tpu as pltpu
from jax.experimental.pallas import tpu_sc as plsc

N_NODES = 10000
N_EDGES = 160000
HIDDEN = 128
BINS = 128
NUM_GRAPHS = 128
VMIN, VMAX = 0.0, 8.0

E_PARTS = (64000, 96000)
BE = 640
BN = 400
GN = N_NODES // BN

SC_CORES = 2
SC_TILES = 16
SC_WORKERS = SC_CORES * SC_TILES
CHUNK = 80

_mesh = functools.partial(
    plsc.VectorSubcoreMesh, core_axis_name="c", subcore_axis_name="s")


def _silu(x):
    return x * jax.nn.sigmoid(x)


def _sc_gather(xcat, gidx3, dep):
    n_chunks = gidx3.shape[0]
    per_w = n_chunks // SC_WORKERS
    D = xcat.shape[1]
    dt = xcat.dtype

    @functools.partial(
        pl.kernel,
        out_type=jax.ShapeDtypeStruct((n_chunks * CHUNK, D), dt),
        mesh=_mesh(),
        scratch_types=[
            pltpu.VMEM((per_w, 1, CHUNK), jnp.int32),
            pltpu.VMEM((CHUNK, D), dt),
            pltpu.VMEM((CHUNK, D), dt),
            pltpu.SemaphoreType.DMA,
            pltpu.SemaphoreType.DMA,
        ],
    )
    def k(x_hbm, idx_hbm, dep_hbm, out_hbm, idx_v3, buf0, buf1, sem0, sem1):
        wid = lax.axis_index("s") * SC_CORES + lax.axis_index("c")
        chunk0 = wid * per_w
        pltpu.sync_copy(idx_hbm.at[pl.ds(chunk0, per_w)], idx_v3)

        def fire(i, buf, sem):
            pltpu.async_copy(x_hbm.at[idx_v3.at[i, 0]], buf, sem)

        def drain(i, buf, sem):
            pltpu.make_async_copy(x_hbm.at[idx_v3.at[i, 0]], buf, sem).wait()

        def wb(i, buf):
            pltpu.sync_copy(buf,
                            out_hbm.at[pl.ds((chunk0 + i) * CHUNK, CHUNK)])

        fire(0, buf0, sem0)
        fire(1, buf1, sem1)

        def body(r, carry):
            i0 = 2 * r
            drain(i0, buf0, sem0)
            wb(i0, buf0)

            @pl.when(i0 + 2 < per_w)
            def _():
                fire(i0 + 2, buf0, sem0)

            drain(i0 + 1, buf1, sem1)
            wb(i0 + 1, buf1)

            @pl.when(i0 + 3 < per_w)
            def _():
                fire(i0 + 3, buf1, sem1)

            return carry

        lax.fori_loop(0, per_w // 2, body, 0)
        if per_w % 2 == 1:
            drain(per_w - 1, buf0, sem0)
            wb(per_w - 1, buf0)

    return k(xcat, gidx3, dep)


ROWS_T = 624
ROWS_TAIL = N_NODES - ROWS_T * SC_TILES


def _sc_scatter(cart_msg, mat_msg, sidx3, zeros_tile, dep):
    n_chunks = sidx3.shape[0]
    per_t = n_chunks // SC_TILES

    @functools.partial(
        pl.kernel,
        out_type=jax.ShapeDtypeStruct((2, N_NODES, HIDDEN), jnp.float32),
        mesh=_mesh(),
        scratch_types=[
            pltpu.VMEM((per_t, 1, CHUNK), jnp.int32),
            pltpu.VMEM((CHUNK, HIDDEN), jnp.float32),
            pltpu.VMEM((CHUNK, HIDDEN), jnp.float32),
            pltpu.VMEM_SHARED((N_NODES, HIDDEN), jnp.float32),
            pltpu.SemaphoreType.DMA,
            pltpu.SemaphoreType.DMA,
        ],
    )
    def k(cm_hbm, mm_hbm, idx_hbm, z_hbm, dep_hbm, out_hbm,
          idx_v3, buf0, buf1, acc_sh, sem0, sem1):
        c = lax.axis_index("c")
        s = lax.axis_index("s")
        chunk0 = s * per_t
        pltpu.sync_copy(z_hbm.at[pl.ds(0, ROWS_T)],
                        acc_sh.at[pl.ds(s * ROWS_T, ROWS_T)])

        @pl.when(s == SC_TILES - 1)
        def _():
            pltpu.sync_copy(z_hbm.at[pl.ds(0, ROWS_TAIL)],
                            acc_sh.at[pl.ds(SC_TILES * ROWS_T, ROWS_TAIL)])

        pltpu.sync_copy(idx_hbm.at[pl.ds(chunk0, per_t)], idx_v3)
        plsc.subcore_barrier()

        def fire(i, buf, sem):
            row0 = (chunk0 + i) * CHUNK

            @pl.when(c == 0)
            def _():
                pltpu.async_copy(cm_hbm.at[pl.ds(row0, CHUNK)], buf, sem)

            @pl.when(c == 1)
            def _():
                pltpu.async_copy(mm_hbm.at[pl.ds(row0, CHUNK)], buf, sem)

        def drain(buf, sem):
            pltpu.make_async_copy(cm_hbm.at[pl.ds(0, CHUNK)], buf, sem).wait()

        def add(i, buf):
            pltpu.sync_copy(buf, acc_sh.at[idx_v3.at[i, 0]], add=True)

        fire(0, buf0, sem0)
        fire(1, buf1, sem1)

        def body(r, carry):
            i0 = 2 * r
            drain(buf0, sem0)
            add(i0, buf0)

            @pl.when(i0 + 2 < per_t)
            def _():
                fire(i0 + 2, buf0, sem0)

            drain(buf1, sem1)
            add(i0 + 1, buf1)

            @pl.when(i0 + 3 < per_t)
            def _():
                fire(i0 + 3, buf1, sem1)

            return carry

        lax.fori_loop(0, per_t // 2, body, 0)
        if per_t % 2 == 1:
            drain(buf0, sem0)
            add(per_t - 1, buf0)
        plsc.subcore_barrier()

        def wb(o):
            pltpu.sync_copy(acc_sh.at[pl.ds(s * ROWS_T, ROWS_T)],
                            out_hbm.at[o, pl.ds(s * ROWS_T, ROWS_T)])

            @pl.when(s == SC_TILES - 1)
            def _():
                pltpu.sync_copy(
                    acc_sh.at[pl.ds(SC_TILES * ROWS_T, ROWS_TAIL)],
                    out_hbm.at[o, pl.ds(SC_TILES * ROWS_T, ROWS_TAIL)])

        @pl.when(c == 0)
        def _():
            wb(0)

        @pl.when(c == 1)
        def _():
            wb(1)

    return k(cart_msg, mat_msg, sidx3, zeros_tile, dep)


def _embed_kernel(t_ref, emb_ref, h_ref):
    t = t_ref[0, 0, :]
    oh = (t[:, None] == lax.broadcasted_iota(jnp.int32, (BN, 128), 1))
    h_ref[...] = jnp.dot(oh.astype(jnp.float32), emb_ref[...],
                         preferred_element_type=jnp.float32)


def _embed(x_types, emb_pad):
    t3 = x_types.astype(jnp.int32).reshape(GN, 1, BN)
    return pl.pallas_call(
        _embed_kernel,
        grid=(GN,),
        in_specs=[
            pl.BlockSpec((1, 1, BN), lambda i: (i, 0, 0)),
            pl.BlockSpec((128, 128), lambda i: (0, 0)),
        ],
        out_specs=pl.BlockSpec((BN, HIDDEN), lambda i: (i, 0)),
        out_shape=jax.ShapeDtypeStruct((N_NODES, HIDDEN), jnp.float32),
    )(t3, emb_pad)


def _rbf_kernel(a_ref, w1_ref, b1_ref, w2_ref, b2_ref, e_ref):
    gamma = (BINS - 1) / (VMAX - VMIN)
    a = a_ref[...]
    d = jnp.sqrt(jnp.sum(a * a, axis=1, keepdims=True))
    centers = (lax.broadcasted_iota(jnp.int32, (1, BINS), 1).astype(jnp.float32)
               * ((VMAX - VMIN) / (BINS - 1)) + VMIN)
    rbf = jnp.exp(-gamma * (d - centers) ** 2)
    h = jnp.dot(rbf, w1_ref[...], preferred_element_type=jnp.float32) + b1_ref[...]
    sp = jnp.where(h > 20.0, h, jnp.log(1.0 + jnp.exp(jnp.minimum(h, 20.0))))
    e_ref[...] = (jnp.dot(sp, w2_ref[...], preferred_element_type=jnp.float32)
                  + b2_ref[...])


def _edge_feats(ea_pad, w1, b1, w2, b2):
    n_e = ea_pad.shape[0]
    return pl.pallas_call(
        _rbf_kernel,
        grid=(n_e // BE,),
        in_specs=[
            pl.BlockSpec((BE, 8), lambda i: (i, 0)),
            pl.BlockSpec((BINS, HIDDEN), lambda i: (0, 0)),
            pl.BlockSpec((1, HIDDEN), lambda i: (0, 0)),
            pl.BlockSpec((HIDDEN, HIDDEN), lambda i: (0, 0)),
            pl.BlockSpec((1, HIDDEN), lambda i: (0, 0)),
        ],
        out_specs=pl.BlockSpec((BE, HIDDEN), lambda i: (i, 0)),
        out_shape=jax.ShapeDtypeStruct((n_e, HIDDEN), jnp.float32),
    )(ea_pad, w1, b1, w2, b2)


def _edge_kernel(xgd_ref, xgs_ref, ec_ref, e0_ref,
                 wcart_ref, bcart_ref, g2_ref, a2_ref, b2c_ref, bne_ref,
                 wqk_ref, bqk_ref, wkk_ref, bk_ref,
                 we_ref, bea_ref, wv_ref, bv_ref, m3_ref,
                 wml_ref, bml_ref, aln_ref, mln_ref,
                 enew_ref, cmsg_ref, mmsg_ref):
    f32 = jnp.float32
    xcd = xgd_ref[:, 0:HIDDEN]
    xmd = xgd_ref[:, HIDDEN:2 * HIDDEN]
    xcs = xgs_ref[:, 0:HIDDEN]
    xms = xgs_ref[:, HIDDEN:2 * HIDDEN]
    ec = ec_ref[...]
    e0 = e0_ref[...]

    z = jnp.concatenate([xcd, xcs, ec], axis=1)
    pre = jnp.dot(z, wcart_ref[...], preferred_element_type=f32) + bcart_ref[...]
    sg = _silu(pre[:, 0:HIDDEN])
    sm = _silu(pre[:, HIDDEN:2 * HIDDEN])
    e_ij = (jnp.dot(sg, g2_ref[...], preferred_element_type=f32)
            + b2c_ref[:, 0:HIDDEN])
    m = (jnp.dot(sm, a2_ref[...], preferred_element_type=f32)
         + b2c_ref[:, HIDDEN:2 * HIDDEN])
    cmsg_ref[...] = jax.nn.sigmoid(e_ij) * m
    enew_ref[...] = _silu(e_ij * bne_ref[0:1, :] + bne_ref[1:2, :]) + ec

    qk = jnp.dot(xmd, wqk_ref[...], preferred_element_type=f32) + bqk_ref[...]
    q_d = qk[:, 0:HIDDEN]
    k_d = qk[:, HIDDEN:2 * HIDDEN]
    k_s = jnp.dot(xms, wkk_ref[...], preferred_element_type=f32) + bk_ref[...]
    ea = jnp.dot(e0, we_ref[...], preferred_element_type=f32) + bea_ref[...]
    scale = 1.0 / np.sqrt(3.0 * HIDDEN).astype(np.float32)
    alpha = jnp.concatenate([q_d * k_d, q_d * k_s, q_d * ea], axis=1) * scale
    mu = jnp.mean(alpha, axis=1, keepdims=True)
    var = jnp.mean((alpha - mu) ** 2, axis=1, keepdims=True)
    aln = ((alpha - mu) / jnp.sqrt(var + 1e-5) * aln_ref[0:1, :]
           + aln_ref[1:2, :])
    sig = jax.nn.sigmoid(aln)
    xm2 = jnp.concatenate([xmd, xms], axis=1)
    vpart = jnp.dot(xm2, wv_ref[...], preferred_element_type=f32) + bv_ref[...]
    msg = (vpart + jnp.dot(ea, m3_ref[...], preferred_element_type=f32)) * sig
    h2 = jnp.dot(msg, wml_ref[...], preferred_element_type=f32) + bml_ref[...]
    mu2 = jnp.mean(h2, axis=1, keepdims=True)
    var2 = jnp.mean((h2 - mu2) ** 2, axis=1, keepdims=True)
    mmsg_ref[...] = ((h2 - mu2) / jnp.sqrt(var2 + 1e-5) * mln_ref[0:1, :]
                     + mln_ref[1:2, :])


def _edge_layer(xg, ec, e0, w):
    ge = ec.shape[0] // BE
    full = lambda shape: pl.BlockSpec(shape, lambda i: (0, 0))
    return pl.pallas_call(
        _edge_kernel,
        grid=(ge,),
        in_specs=[
            pl.BlockSpec((BE, 2 * HIDDEN), lambda i: (i, 0)),
            pl.BlockSpec((BE, 2 * HIDDEN), lambda i: (i + ge, 0)),
            pl.BlockSpec((BE, HIDDEN), lambda i: (i, 0)),
            pl.BlockSpec((BE, HIDDEN), lambda i: (i, 0)),
            full((3 * HIDDEN, 2 * HIDDEN)), full((1, 2 * HIDDEN)),
            full((HIDDEN, HIDDEN)), full((HIDDEN, HIDDEN)),
            full((1, 2 * HIDDEN)), full((2, HIDDEN)),
            full((HIDDEN, 2 * HIDDEN)), full((1, 2 * HIDDEN)),
            full((HIDDEN, HIDDEN)), full((1, HIDDEN)),
            full((HIDDEN, HIDDEN)), full((1, HIDDEN)),
            full((2 * HIDDEN, 3 * HIDDEN)), full((1, 3 * HIDDEN)),
            full((HIDDEN, 3 * HIDDEN)),
            full((3 * HIDDEN, HIDDEN)), full((1, HIDDEN)),
            full((2, 3 * HIDDEN)), full((2, HIDDEN)),
        ],
        out_specs=[
            pl.BlockSpec((BE, HIDDEN), lambda i: (i, 0)),
            pl.BlockSpec((BE, HIDDEN), lambda i: (i, 0)),
            pl.BlockSpec((BE, HIDDEN), lambda i: (i, 0)),
        ],
        out_shape=[
            jax.ShapeDtypeStruct((ec.shape[0], HIDDEN), jnp.float32),
            jax.ShapeDtypeStruct((ec.shape[0], HIDDEN), jnp.float32),
            jax.ShapeDtypeStruct((ec.shape[0], HIDDEN), jnp.float32),
        ],
    )(xg, xg, ec, e0,
      w["Wcart"], w["bcart"], w["G2"], w["A2"], w["b2c"], w["bne"],
      w["Wqk"], w["bqk"], w["Wkk"], w["bk"],
      w["We"], w["bea"], w["WV"], w["bV"], w["M3"],
      w["Wml"], w["bml"], w["aln"], w["mln"])


def _node_kernel(xc_ref, xm_ref, aggc0_ref, aggm0_ref, aggc1_ref, aggm1_ref,
                 bnx_ref, ws_ref, bs_ref, wb_ref, wmix_ref, bmix_ref,
                 xcn_ref, xmn_ref):
    f32 = jnp.float32
    xc_old = xc_ref[...]
    xm_old = xm_ref[...]
    agg = aggc0_ref[0] + aggc1_ref[0]
    out = aggm0_ref[0] + aggm1_ref[0]
    x_c = _silu(agg * bnx_ref[0:1, :] + bnx_ref[1:2, :]) + xc_old
    x_r = jnp.dot(xm_old, ws_ref[...], preferred_element_type=f32) + bs_ref[0:1, :]
    bpre = (jnp.sum(out * wb_ref[0:1, :], axis=1, keepdims=True)
            + jnp.sum(x_r * wb_ref[1:2, :], axis=1, keepdims=True)
            + wb_ref[2, 0])
    beta = jax.nn.sigmoid(bpre)
    x_m = beta * x_r + (1.0 - beta) * out
    z = jnp.concatenate([x_c, x_m], axis=1)
    gate = jax.nn.sigmoid(
        jnp.dot(z, wmix_ref[...], preferred_element_type=f32) + bmix_ref[...])
    x_out = gate * x_c + (1.0 - gate) * x_m
    xcn_ref[...] = x_out + x_c
    xmn_ref[...] = x_out + x_m


def _node_layer(x_cart, x_mat, agg2a, agg2b, w):
    full = lambda shape: pl.BlockSpec(shape, lambda i: tuple(0 for _ in shape))
    return pl.pallas_call(
        _node_kernel,
        grid=(GN,),
        in_specs=[
            pl.BlockSpec((BN, HIDDEN), lambda i: (i, 0)),
            pl.BlockSpec((BN, HIDDEN), lambda i: (i, 0)),
            pl.BlockSpec((1, BN, HIDDEN), lambda i: (0, i, 0)),
            pl.BlockSpec((1, BN, HIDDEN), lambda i: (1, i, 0)),
            pl.BlockSpec((1, BN, HIDDEN), lambda i: (0, i, 0)),
            pl.BlockSpec((1, BN, HIDDEN), lambda i: (1, i, 0)),
            full((2, HIDDEN)),
            full((HIDDEN, HIDDEN)), full((1, HIDDEN)),
            full((3, HIDDEN)),
            full((2 * HIDDEN, HIDDEN)), full((1, HIDDEN)),
        ],
        out_specs=[
            pl.BlockSpec((BN, HIDDEN), lambda i: (i, 0)),
            pl.BlockSpec((BN, HIDDEN), lambda i: (i, 0)),
        ],
        out_shape=[
            jax.ShapeDtypeStruct((N_NODES, HIDDEN), jnp.float32),
            jax.ShapeDtypeStruct((N_NODES, HIDDEN), jnp.float32),
        ],
    )(x_cart, x_mat, agg2a, agg2a, agg2b, agg2b,
      w["bnx"], w["Ws"], w["bs"], w["wb"], w["Wmix"], w["bmix"])


def _readout_kernel(xc_ref, xm_ref, b_ref,
                    w1_ref, b1_ref, w2r_ref, misc_ref,
                    out_ref, sums_ref):
    i = pl.program_id(0)

    @pl.when(i == 0)
    def _():
        sums_ref[...] = jnp.zeros_like(sums_ref)

    xf = (xc_ref[...] + xm_ref[...]) * 0.5
    bids = b_ref[0, 0, :]
    oh = (lax.broadcasted_iota(jnp.int32, (NUM_GRAPHS, BN), 0)
          == bids[None, :]).astype(jnp.float32)
    ones = jnp.ones((BN, HIDDEN), jnp.float32)
    xa = jnp.concatenate([xf, ones], axis=1)
    sums_ref[...] += jnp.dot(oh, xa, preferred_element_type=jnp.float32)

    @pl.when(i == GN - 1)
    def _():
        sums = sums_ref[:, 0:HIDDEN]
        counts = sums_ref[:, HIDDEN:HIDDEN + 1]
        feats = sums / jnp.maximum(counts, 1.0)
        h = _silu(jnp.dot(feats, w1_ref[...], preferred_element_type=jnp.float32)
                  + b1_ref[...])
        o = lax.dot_general(w2r_ref[...], h, (((1,), (1,)), ((), ())))
        out_ref[...] = o + misc_ref[0, 0]


def _readout(x_cart, x_mat, batch_ids, w1, b1, w2r, misc):
    b3 = batch_ids.astype(jnp.int32).reshape(GN, 1, BN)
    full = lambda shape: pl.BlockSpec(shape, lambda i: tuple(0 for _ in shape))
    return pl.pallas_call(
        _readout_kernel,
        grid=(GN,),
        in_specs=[
            pl.BlockSpec((BN, HIDDEN), lambda i: (i, 0)),
            pl.BlockSpec((BN, HIDDEN), lambda i: (i, 0)),
            pl.BlockSpec((1, 1, BN), lambda i: (i, 0, 0)),
            full((HIDDEN, HIDDEN)), full((1, HIDDEN)),
            full((1, HIDDEN)), full((1, HIDDEN)),
        ],
        out_specs=pl.BlockSpec((1, NUM_GRAPHS), lambda i: (0, 0)),
        out_shape=jax.ShapeDtypeStruct((1, NUM_GRAPHS), jnp.float32),
        scratch_shapes=[pltpu.VMEM((NUM_GRAPHS, 2 * HIDDEN), jnp.float32)],
    )(x_cart, x_mat, b3, w1, b1, w2r, misc)


def _fold_layer(lp):
    r1 = lambda b: b.reshape(1, -1)
    Wmu = lp["mat_msg_update"]["W"]
    M1, M2, M3 = Wmu[0:128], Wmu[128:256], Wmu[256:384]
    Wv, bv = lp["mat_v"]["W"], lp["mat_v"]["b"]
    Wbeta = lp["mat_beta"]["W"][:, 0]
    wb = jnp.stack([
        Wbeta[0:128] + Wbeta[256:384],
        Wbeta[128:256] - Wbeta[256:384],
        jnp.full((128,), lp["mat_beta"]["b"][0]),
    ])
    return {
        "Wcart": jnp.concatenate([lp["cart_gate1"]["W"],
                                  lp["cart_aggr1"]["W"]], axis=1),
        "bcart": jnp.concatenate([lp["cart_gate1"]["b"],
                                  lp["cart_aggr1"]["b"]]).reshape(1, -1),
        "G2": lp["cart_gate2"]["W"], "A2": lp["cart_aggr2"]["W"],
        "b2c": jnp.concatenate([lp["cart_gate2"]["b"],
                                lp["cart_aggr2"]["b"]]).reshape(1, -1),
        "bne": jnp.stack([lp["cart_bn_e"]["w"], lp["cart_bn_e"]["b"]]),
        "Wqk": jnp.concatenate([lp["mat_q"]["W"], lp["mat_k"]["W"]], axis=1),
        "bqk": jnp.concatenate([lp["mat_q"]["b"],
                                lp["mat_k"]["b"]]).reshape(1, -1),
        "Wkk": lp["mat_k"]["W"], "bk": r1(lp["mat_k"]["b"]),
        "We": lp["mat_edge"]["W"], "bea": r1(lp["mat_edge"]["b"]),
        "WV": jnp.concatenate([Wv @ M1, Wv @ M2], axis=0),
        "bV": (bv @ M1 + bv @ M2 + lp["mat_msg_update"]["b"]).reshape(1, -1),
        "M3": M3,
        "Wml": lp["mat_msg_lin"]["W"], "bml": r1(lp["mat_msg_lin"]["b"]),
        "aln": jnp.stack([lp["mat_alpha_ln"]["w"], lp["mat_alpha_ln"]["b"]]),
        "mln": jnp.stack([lp["mat_msg_ln"]["w"], lp["mat_msg_ln"]["b"]]),
        "bnx": jnp.stack([lp["cart_bn_x"]["w"], lp["cart_bn_x"]["b"]]),
        "Ws": lp["mat_skip"]["W"], "bs": r1(lp["mat_skip"]["b"]),
        "wb": wb,
        "Wmix": lp["mix_gate"]["W"], "bmix": r1(lp["mix_gate"]["b"]),
    }


def kernel(edge_attr, params, x_types, edge_index, batch_ids):
    src = edge_index[0].astype(jnp.int32)
    dst = edge_index[1].astype(jnp.int32)

    emb_pad = jnp.zeros((128, HIDDEN), jnp.float32).at[:119].set(
        params["atom_emb"])
    h = _embed(x_types, emb_pad)

    e0s, gidxs, sidxs = [], [], []
    off = 0
    for n_e in E_PARTS:
        sl = slice(off, off + n_e)
        off += n_e
        ea_pad = jnp.concatenate(
            [edge_attr[sl], jnp.zeros((n_e, 5), jnp.float32)], axis=1)
        e0s.append(_edge_feats(ea_pad, params["rbf_lin1"]["W"],
                               params["rbf_lin1"]["b"].reshape(1, -1),
                               params["rbf_lin2"]["W"],
                               params["rbf_lin2"]["b"].reshape(1, -1)))
        gidxs.append(jnp.concatenate([dst[sl], src[sl]]).reshape(-1, 1, CHUNK))
        sidxs.append(dst[sl].reshape(-1, 1, CHUNK))
    zeros_tile = jnp.zeros((ROWS_T, HIDDEN), jnp.float32)

    x_cart, x_mat = h, h
    ec = list(e0s)
    for lp in params["layers"]:
        w = _fold_layer(lp)
        xcat = jnp.concatenate([x_cart, x_mat], axis=1)
        xg_a = _sc_gather(xcat, gidxs[0], zeros_tile)
        xg_b = _sc_gather(xcat, gidxs[1], xg_a)
        ec_a, cmsg_a, mmsg_a = _edge_layer(xg_a, ec[0], e0s[0], w)
        agg_a = _sc_scatter(cmsg_a, mmsg_a, sidxs[0], zeros_tile, xg_b)
        ec_b, cmsg_b, mmsg_b = _edge_layer(xg_b, ec[1], e0s[1], w)
        agg_b = _sc_scatter(cmsg_b, mmsg_b, sidxs[1], zeros_tile, agg_a)
        ec = [ec_a, ec_b]
        x_cart, x_mat = _node_layer(x_cart, x_mat, agg_a, agg_b, w)

    out = _readout(x_cart, x_mat, batch_ids,
                   params["fc1"]["W"], params["fc1"]["b"].reshape(1, -1),
                   params["fc2"]["W"].T.reshape(1, -1),
                   jnp.full((1, HIDDEN), params["fc2"]["b"][0]))
    return out.reshape(NUM_GRAPHS)
